# baseline (speedup 1.0000x reference)
import jax
import jax.numpy as jnp
from jax import lax
from jax.experimental import pallas as pl
from jax.experimental.pallas import tpu as pltpu
from jax.experimental.pallas import tpu_sc as plsc

NUM_HIDDENS = 384
EMBED_DIM = 256
N_EMBED = 1024
KL_WEIGHT = 0.0005
B = 8
HW = 576

_NC = 2
_NS = 16
_NW = _NC * _NS
_D_PER_W = EMBED_DIM // _NW
_LANES = 16
_CHUNKS = HW // _LANES


def _tc_body(z_ref, w_ref, b_ref, ind_ref, loss_ref):
    b = pl.program_id(0)
    zb = z_ref[0].reshape(NUM_HIDDENS, HW)
    logits = jax.lax.dot_general(
        w_ref[...], zb, (((1,), (0,)), ((), ())),
        preferred_element_type=jnp.float32)
    logits = logits + b_ref[...]
    m = jnp.max(logits, axis=0, keepdims=True)
    t = logits - m
    e = jnp.exp(t)
    zsum = jnp.sum(e, axis=0, keepdims=True)
    sxm = jnp.sum(e * t, axis=0, keepdims=True)
    kl_cols = jnp.log(jnp.float32(N_EMBED)) + sxm / zsum - jnp.log(zsum)
    kl = jnp.sum(kl_cols)
    rows = jax.lax.broadcasted_iota(jnp.int32, (N_EMBED, HW), 0)
    ind = jnp.min(jnp.where(e == 1.0, rows, jnp.int32(1 << 30)), axis=0)
    ind_ref[0, 0, :] = ind

    @pl.when(b == 0)
    def _():
        loss_ref[...] = jnp.zeros((1, 1), jnp.float32)

    loss_ref[...] += jnp.full((1, 1), kl, jnp.float32)

    @pl.when(b == B - 1)
    def _():
        loss_ref[...] *= jnp.float32(KL_WEIGHT / (B * HW))


def _sc_body(et_hbm, ind_hbm, out_hbm, tab_v, idx_v, obuf_v, sem):
    wid = lax.axis_index("s") * _NC + lax.axis_index("c")
    d0 = wid * _D_PER_W
    pltpu.sync_copy(et_hbm.at[pl.ds(d0 * N_EMBED, _D_PER_W * N_EMBED)], tab_v)
    pltpu.sync_copy(ind_hbm, idx_v)

    @plsc.parallel_loop(0, B * _CHUNKS, 1, unroll=2)
    def _chunk(i):
        idx = idx_v[pl.ds(i * _LANES, _LANES)]
        b = i // _CHUNKS
        c = i - b * _CHUNKS
        for dl in range(_D_PER_W):
            obuf_v[b, dl, pl.ds(c * _LANES, _LANES)] = plsc.load_gather(
                tab_v, [idx + jnp.int32(dl * N_EMBED)])

    copies = []
    for b in range(B):
        cp = pltpu.make_async_copy(
            obuf_v.at[pl.ds(b, 1)],
            out_hbm.at[pl.ds(b, 1), pl.ds(d0, _D_PER_W)], sem)
        cp.start()
        copies.append(cp)
    for cp in copies:
        cp.wait()


_sc_gather = pl.kernel(
    _sc_body,
    out_type=jax.ShapeDtypeStruct((B, EMBED_DIM, HW), jnp.float32),
    mesh=plsc.VectorSubcoreMesh(core_axis_name="c", subcore_axis_name="s"),
    scratch_types=[
        pltpu.VMEM((_D_PER_W * N_EMBED,), jnp.float32),
        pltpu.VMEM((B * HW,), jnp.int32),
        pltpu.VMEM((B, _D_PER_W, HW), jnp.float32),
        pltpu.SemaphoreType.DMA,
    ],
    compiler_params=pltpu.CompilerParams(needs_layout_passes=False),
)


@jax.jit
def kernel(z, W_proj, b_proj, embed_w):
    b2 = b_proj.reshape(N_EMBED, 1)
    embed_wT = embed_w.T
    ind3, loss = pl.pallas_call(
        _tc_body,
        grid=(B,),
        in_specs=[
            pl.BlockSpec((1, NUM_HIDDENS, 24, 24), lambda b: (b, 0, 0, 0)),
            pl.BlockSpec((N_EMBED, NUM_HIDDENS), lambda b: (0, 0)),
            pl.BlockSpec((N_EMBED, 1), lambda b: (0, 0)),
        ],
        out_specs=[
            pl.BlockSpec((1, 1, HW), lambda b: (b, 0, 0)),
            pl.BlockSpec((1, 1), lambda b: (0, 0)),
        ],
        out_shape=[
            jax.ShapeDtypeStruct((B, 1, HW), jnp.int32),
            jax.ShapeDtypeStruct((1, 1), jnp.float32),
        ],
    )(z, W_proj, b2)
    ind_flat = ind3.reshape(B * HW)
    zq = _sc_gather(embed_wT.reshape(-1), ind_flat)
    z_q = zq.reshape(B, EMBED_DIM, 24, 24)
    ind = ind3.reshape(B, 24, 24)
    prior_loss = loss[0, 0]
    return (z_q, prior_loss, ind)


# speedup vs baseline: 1.2011x; 1.2011x over previous
import jax
import jax.numpy as jnp
from jax import lax
from jax.experimental import pallas as pl
from jax.experimental.pallas import tpu as pltpu
from jax.experimental.pallas import tpu_sc as plsc

NUM_HIDDENS = 384
EMBED_DIM = 256
N_EMBED = 1024
KL_WEIGHT = 0.0005
B = 8
HW = 576
BH = B // 2

_NC = 2
_NS = 16
_NW = _NC * _NS
_D_PER_W = EMBED_DIM // _NW
_LANES = 16
_CHUNKS = HW // _LANES


def _tc_body(z_ref, w_ref, b_ref, ind_ref, loss_ref):
    b = pl.program_id(0)
    zb = z_ref[0]
    logits = jax.lax.dot_general(
        w_ref[...], zb, (((1,), (0,)), ((), ())),
        preferred_element_type=jnp.float32)
    logits = logits + b_ref[...]
    m = jnp.max(logits, axis=0, keepdims=True)
    t = logits - m
    e = jnp.exp(t)
    zsum = jnp.sum(e, axis=0, keepdims=True)
    sxm = jnp.sum(e * t, axis=0, keepdims=True)
    kl_cols = jnp.log(jnp.float32(N_EMBED)) + sxm / zsum - jnp.log(zsum)
    kl = jnp.sum(kl_cols)
    rows = jax.lax.broadcasted_iota(jnp.int32, (N_EMBED, HW), 0)
    ind = jnp.min(jnp.where(e == 1.0, rows, jnp.int32(1 << 30)), axis=0)
    ind_ref[0, 0, :] = ind

    @pl.when(b == 0)
    def _():
        loss_ref[...] = jnp.zeros((1, 1), jnp.float32)

    loss_ref[...] += jnp.full((1, 1), kl, jnp.float32)


def _sc_body(et_hbm, ind_hbm, out_hbm, tab_v, idx_v, obuf_v, sem):
    wid = lax.axis_index("s") * _NC + lax.axis_index("c")
    d0 = wid * _D_PER_W
    pltpu.sync_copy(et_hbm.at[pl.ds(d0 * N_EMBED, _D_PER_W * N_EMBED)], tab_v)
    pltpu.sync_copy(ind_hbm, idx_v)

    @plsc.parallel_loop(0, BH * _CHUNKS, 1, unroll=2)
    def _chunk(i):
        idx = idx_v[pl.ds(i * _LANES, _LANES)]
        b = i // _CHUNKS
        c = i - b * _CHUNKS
        for dl in range(_D_PER_W):
            obuf_v[b, dl, pl.ds(c * _LANES, _LANES)] = plsc.load_gather(
                tab_v, [idx + jnp.int32(dl * N_EMBED)])

    copies = []
    for b in range(BH):
        cp = pltpu.make_async_copy(
            obuf_v.at[pl.ds(b, 1)],
            out_hbm.at[pl.ds(b, 1), pl.ds(d0, _D_PER_W)], sem)
        cp.start()
        copies.append(cp)
    for cp in copies:
        cp.wait()


_sc_gather = pl.kernel(
    _sc_body,
    out_type=jax.ShapeDtypeStruct((BH, EMBED_DIM, HW), jnp.float32),
    mesh=plsc.VectorSubcoreMesh(core_axis_name="c", subcore_axis_name="s"),
    scratch_types=[
        pltpu.VMEM((_D_PER_W * N_EMBED,), jnp.float32),
        pltpu.VMEM((BH * HW,), jnp.int32),
        pltpu.VMEM((BH, _D_PER_W, HW), jnp.float32),
        pltpu.SemaphoreType.DMA,
    ],
    compiler_params=pltpu.CompilerParams(needs_layout_passes=False),
)


def _tc_half(zf_half, W_proj, b2):
    return pl.pallas_call(
        _tc_body,
        grid=(BH,),
        in_specs=[
            pl.BlockSpec((1, NUM_HIDDENS, HW), lambda b: (b, 0, 0)),
            pl.BlockSpec((N_EMBED, NUM_HIDDENS), lambda b: (0, 0)),
            pl.BlockSpec((N_EMBED, 1), lambda b: (0, 0)),
        ],
        out_specs=[
            pl.BlockSpec((1, 1, HW), lambda b: (b, 0, 0)),
            pl.BlockSpec((1, 1), lambda b: (0, 0)),
        ],
        out_shape=[
            jax.ShapeDtypeStruct((BH, 1, HW), jnp.int32),
            jax.ShapeDtypeStruct((1, 1), jnp.float32),
        ],
    )(zf_half, W_proj, b2)


@jax.jit
def kernel(z, W_proj, b_proj, embed_w):
    zf = z.reshape(B, NUM_HIDDENS, HW)
    b2 = b_proj.reshape(N_EMBED, 1)
    embed_wT = embed_w.T.reshape(-1)
    ind_a, loss_a = _tc_half(zf[:BH], W_proj, b2)
    zq_a = _sc_gather(embed_wT, ind_a.reshape(BH * HW))
    ind_b, loss_b = _tc_half(zf[BH:], W_proj, b2)
    zq_b = _sc_gather(embed_wT, ind_b.reshape(BH * HW))
    z_q = jnp.concatenate([zq_a, zq_b], axis=0).reshape(B, EMBED_DIM, 24, 24)
    ind = jnp.concatenate([ind_a, ind_b], axis=0).reshape(B, 24, 24)
    prior_loss = (loss_a[0, 0] + loss_b[0, 0]) * jnp.float32(
        KL_WEIGHT / (B * HW))
    return (z_q, prior_loss, ind)


# speedup vs baseline: 1.4275x; 1.1885x over previous
import jax
import jax.numpy as jnp
from jax import lax
from jax.experimental import pallas as pl
from jax.experimental.pallas import tpu as pltpu
from jax.experimental.pallas import tpu_sc as plsc

NUM_HIDDENS = 384
EMBED_DIM = 256
N_EMBED = 1024
KL_WEIGHT = 0.0005
B = 8
HW = 576

_NC = 2
_NS = 16
_NW = _NC * _NS
_D_PER_W = EMBED_DIM // _NW
_LANES = 16
_CHUNKS = HW // _LANES


def _tc_body(z_ref, w_ref, b_ref, ind_ref, loss_ref):
    b = pl.program_id(0)
    zb = z_ref[0]
    logits = jax.lax.dot_general(
        w_ref[...], zb, (((1,), (0,)), ((), ())),
        preferred_element_type=jnp.float32)
    logits = logits + b_ref[...]
    m = jnp.max(logits, axis=0, keepdims=True)
    t = logits - m
    e = jnp.exp(t)
    zsum = jnp.sum(e, axis=0, keepdims=True)
    sxm = jnp.sum(e * t, axis=0, keepdims=True)
    kl_cols = jnp.log(jnp.float32(N_EMBED)) + sxm / zsum - jnp.log(zsum)
    kl = jnp.sum(kl_cols)
    rows = jax.lax.broadcasted_iota(jnp.int32, (N_EMBED, HW), 0)
    ind = jnp.min(jnp.where(e == 1.0, rows, jnp.int32(1 << 30)), axis=0)
    ind_ref[0, 0, :] = ind

    @pl.when(b == 0)
    def _():
        loss_ref[...] = jnp.zeros((1, 1), jnp.float32)

    loss_ref[...] += jnp.full((1, 1), kl, jnp.float32)

    @pl.when(b == B - 1)
    def _():
        loss_ref[...] *= jnp.float32(KL_WEIGHT / (B * HW))


def _sc_body(et_hbm, ind_hbm, out_hbm, tab_v, idx_v, obuf_v, sem, sem2):
    wid = lax.axis_index("s") * _NC + lax.axis_index("c")
    d0 = wid * _D_PER_W
    cp_tab = pltpu.make_async_copy(
        et_hbm.at[pl.ds(d0 * N_EMBED, _D_PER_W * N_EMBED)], tab_v, sem2)
    cp_idx = pltpu.make_async_copy(ind_hbm, idx_v, sem2)
    cp_tab.start()
    cp_idx.start()
    cp_tab.wait()
    cp_idx.wait()

    @plsc.parallel_loop(0, B * _CHUNKS, 1, unroll=4)
    def _chunk(i):
        b = i // _CHUNKS
        c = i - b * _CHUNKS
        idx = idx_v[b, 0, pl.ds(c * _LANES, _LANES)]
        for dl in range(_D_PER_W):
            obuf_v[b, dl, pl.ds(c * _LANES, _LANES)] = plsc.load_gather(
                tab_v, [idx + jnp.int32(dl * N_EMBED)])

    copies = []
    for b in range(B):
        cp = pltpu.make_async_copy(
            obuf_v.at[pl.ds(b, 1)],
            out_hbm.at[pl.ds(b, 1), pl.ds(d0, _D_PER_W)], sem)
        cp.start()
        copies.append(cp)
    for cp in copies:
        cp.wait()


_sc_gather = pl.kernel(
    _sc_body,
    out_type=jax.ShapeDtypeStruct((B, EMBED_DIM, HW), jnp.float32),
    mesh=plsc.VectorSubcoreMesh(core_axis_name="c", subcore_axis_name="s"),
    scratch_types=[
        pltpu.VMEM((_D_PER_W * N_EMBED,), jnp.float32),
        pltpu.VMEM((B, 1, HW), jnp.int32),
        pltpu.VMEM((B, _D_PER_W, HW), jnp.float32),
        pltpu.SemaphoreType.DMA,
        pltpu.SemaphoreType.DMA,
    ],
    compiler_params=pltpu.CompilerParams(needs_layout_passes=False),
)


@jax.jit
def kernel(z, W_proj, b_proj, embed_w):
    zf = z.reshape(B, NUM_HIDDENS, HW)
    b2 = b_proj.reshape(N_EMBED, 1)
    embed_wT = embed_w.T.reshape(-1)
    ind3, loss = pl.pallas_call(
        _tc_body,
        grid=(B,),
        in_specs=[
            pl.BlockSpec((1, NUM_HIDDENS, HW), lambda b: (b, 0, 0)),
            pl.BlockSpec((N_EMBED, NUM_HIDDENS), lambda b: (0, 0)),
            pl.BlockSpec((N_EMBED, 1), lambda b: (0, 0)),
        ],
        out_specs=[
            pl.BlockSpec((1, 1, HW), lambda b: (b, 0, 0)),
            pl.BlockSpec((1, 1), lambda b: (0, 0)),
        ],
        out_shape=[
            jax.ShapeDtypeStruct((B, 1, HW), jnp.int32),
            jax.ShapeDtypeStruct((1, 1), jnp.float32),
        ],
    )(zf, W_proj, b2)
    zq = _sc_gather(embed_wT, ind3)
    z_q = zq.reshape(B, EMBED_DIM, 24, 24)
    ind = ind3.reshape(B, 24, 24)
    prior_loss = loss[0, 0]
    return (z_q, prior_loss, ind)


# speedup vs baseline: 1.4306x; 1.0021x over previous
import jax
import jax.numpy as jnp
from jax import lax
from jax.experimental import pallas as pl
from jax.experimental.pallas import tpu as pltpu
from jax.experimental.pallas import tpu_sc as plsc

NUM_HIDDENS = 384
EMBED_DIM = 256
N_EMBED = 1024
KL_WEIGHT = 0.0005
B = 8
HW = 576

_NC = 2
_NS = 16
_NW = _NC * _NS
_D_PER_W = EMBED_DIM // _NW
_LANES = 16
_CHUNKS = HW // _LANES


def _tc_body(z_ref, w_ref, b_ref, ind_ref, loss_ref):
    b = pl.program_id(0)
    zb = z_ref[0]
    logits = jax.lax.dot_general(
        w_ref[...], zb, (((1,), (0,)), ((), ())),
        preferred_element_type=jnp.float32)
    logits = logits + b_ref[...]
    m = jnp.max(logits, axis=0, keepdims=True)
    t = logits - m
    e = jnp.exp(t)
    zsum = jnp.sum(e, axis=0, keepdims=True)
    sxm = jnp.sum(e * t, axis=0, keepdims=True)
    kl_cols = jnp.log(jnp.float32(N_EMBED)) + sxm / zsum - jnp.log(zsum)
    kl = jnp.sum(kl_cols)
    rows = jax.lax.broadcasted_iota(jnp.int32, (N_EMBED, HW), 0)
    ind = jnp.min(jnp.where(e == 1.0, rows, jnp.int32(1 << 30)), axis=0)
    ind_ref[0, 0, :] = ind

    @pl.when(b == 0)
    def _():
        loss_ref[...] = jnp.zeros((1, 1), jnp.float32)

    loss_ref[...] += jnp.full((1, 1), kl, jnp.float32)

    @pl.when(b == B - 1)
    def _():
        loss_ref[...] *= jnp.float32(KL_WEIGHT / (B * HW))


def _sc_body(et_hbm, ind_hbm, out_hbm, tab_v, idx_v, obuf_v, sem, sem2):
    wid = lax.axis_index("s") * _NC + lax.axis_index("c")
    d0 = wid * _D_PER_W
    cp_tab = pltpu.make_async_copy(
        et_hbm.at[pl.ds(d0 * N_EMBED, _D_PER_W * N_EMBED)], tab_v, sem2)
    cp_idx = pltpu.make_async_copy(ind_hbm, idx_v, sem2)
    cp_tab.start()
    cp_idx.start()
    cp_tab.wait()
    cp_idx.wait()

    @plsc.parallel_loop(0, B * _CHUNKS, 1, unroll=4)
    def _chunk(i):
        b = i // _CHUNKS
        c = i - b * _CHUNKS
        idx = idx_v[b, 0, pl.ds(c * _LANES, _LANES)]
        for dl in range(_D_PER_W):
            obuf_v[b, dl, pl.ds(c * _LANES, _LANES)] = plsc.load_gather(
                tab_v, [idx + jnp.int32(dl * N_EMBED)])

    pltpu.sync_copy(obuf_v, out_hbm.at[:, pl.ds(d0, _D_PER_W)])


_sc_gather = pl.kernel(
    _sc_body,
    out_type=jax.ShapeDtypeStruct((B, EMBED_DIM, HW), jnp.float32),
    mesh=plsc.VectorSubcoreMesh(core_axis_name="c", subcore_axis_name="s"),
    scratch_types=[
        pltpu.VMEM((_D_PER_W * N_EMBED,), jnp.float32),
        pltpu.VMEM((B, 1, HW), jnp.int32),
        pltpu.VMEM((B, _D_PER_W, HW), jnp.float32),
        pltpu.SemaphoreType.DMA,
        pltpu.SemaphoreType.DMA,
    ],
    compiler_params=pltpu.CompilerParams(needs_layout_passes=False),
)


@jax.jit
def kernel(z, W_proj, b_proj, embed_w):
    zf = z.reshape(B, NUM_HIDDENS, HW)
    b2 = b_proj.reshape(N_EMBED, 1)
    embed_wT = embed_w.T.reshape(-1)
    ind3, loss = pl.pallas_call(
        _tc_body,
        grid=(B,),
        in_specs=[
            pl.BlockSpec((1, NUM_HIDDENS, HW), lambda b: (b, 0, 0)),
            pl.BlockSpec((N_EMBED, NUM_HIDDENS), lambda b: (0, 0)),
            pl.BlockSpec((N_EMBED, 1), lambda b: (0, 0)),
        ],
        out_specs=[
            pl.BlockSpec((1, 1, HW), lambda b: (b, 0, 0)),
            pl.BlockSpec((1, 1), lambda b: (0, 0)),
        ],
        out_shape=[
            jax.ShapeDtypeStruct((B, 1, HW), jnp.int32),
            jax.ShapeDtypeStruct((1, 1), jnp.float32),
        ],
    )(zf, W_proj, b2)
    zq = _sc_gather(embed_wT, ind3)
    z_q = zq.reshape(B, EMBED_DIM, 24, 24)
    ind = ind3.reshape(B, 24, 24)
    prior_loss = loss[0, 0]
    return (z_q, prior_loss, ind)
